# baseline (speedup 1.0000x reference)
import functools

import jax
import jax.numpy as jnp
from jax import lax
from jax.experimental import pallas as pl
from jax.experimental.pallas import tpu as pltpu
from jax.experimental.pallas import tpu_sc as plsc

_LANES = 128


def _sc_gather(emb128, idx):
    _, d = emb128.shape
    n = idx.shape[0]
    info = plsc.get_sparse_core_info()
    nc, ns = info.num_cores, info.num_subcores
    nw = nc * ns
    assert n % nw == 0 and (n // nw) % 8 == 0
    n_per_w = n // nw
    mesh = plsc.VectorSubcoreMesh(core_axis_name="c", subcore_axis_name="s")

    @functools.partial(
        pl.kernel,
        mesh=mesh,
        out_type=jax.ShapeDtypeStruct((n, d), jnp.float32),
        compiler_params=pltpu.CompilerParams(use_tc_tiling_on_sc=True),
        scratch_types=[
            pltpu.VMEM((n_per_w,), jnp.int32),
            pltpu.VMEM((n_per_w, d), jnp.float32),
            pltpu.SemaphoreType.DMA,
        ],
    )
    def gather_kernel(table_hbm, idx_hbm, out_hbm, idx_v, rows_v, sem):
        wid = lax.axis_index("s") * nc + lax.axis_index("c")
        base = wid * n_per_w
        pltpu.sync_copy(idx_hbm.at[pl.ds(base, n_per_w)], idx_v)
        pltpu.async_copy(table_hbm.at[idx_v], rows_v, sem).wait()
        pltpu.sync_copy(rows_v, out_hbm.at[pl.ds(base, n_per_w)])

    return gather_kernel(emb128, idx)


def _mlp(e_pad, w_h, b_h2, w_o, b_o2, vt=4096):
    bsz, wpad = e_pad.shape
    h, wd = w_h.shape
    window = wpad // _LANES
    d = wd // window
    v = w_o.shape[0]
    grid = (v + vt - 1) // vt

    def body(e_ref, wh_ref, bh_ref, wo_ref, bo_ref, out_ref, h_ref):
        @pl.when(pl.program_id(0) == 0)
        def _():
            e = jnp.concatenate(
                [e_ref[:, w * _LANES:w * _LANES + d] for w in range(window)],
                axis=1)
            acc = lax.dot_general(
                e, wh_ref[...], (((1,), (1,)), ((), ())),
                preferred_element_type=jnp.float32)
            h_ref[...] = jnp.tanh(acc + bh_ref[...])

        out_ref[...] = lax.dot_general(
            h_ref[...], wo_ref[...], (((1,), (1,)), ((), ())),
            preferred_element_type=jnp.float32) + bo_ref[...]

    return pl.pallas_call(
        body,
        grid=(grid,),
        in_specs=[
            pl.BlockSpec((bsz, wpad), lambda j: (0, 0)),
            pl.BlockSpec((h, wd), lambda j: (0, 0)),
            pl.BlockSpec((1, h), lambda j: (0, 0)),
            pl.BlockSpec((vt, h), lambda j: (j, 0)),
            pl.BlockSpec((1, vt), lambda j: (0, j)),
        ],
        out_specs=pl.BlockSpec((bsz, vt), lambda j: (0, j)),
        out_shape=jax.ShapeDtypeStruct((bsz, v), jnp.float32),
        scratch_shapes=[pltpu.VMEM((bsz, h), jnp.float32)],
    )(e_pad, w_h, b_h2, w_o, b_o2)


def kernel(x, emb, W_h, b_h, W_o, b_o):
    batch, window = x.shape
    emb128 = jnp.pad(emb, ((0, 0), (0, _LANES - emb.shape[1])))
    idx = x.reshape(-1).astype(jnp.int32)
    rows = _sc_gather(emb128, idx)
    e_pad = rows.reshape(batch, window * _LANES)
    return _mlp(e_pad, W_h, b_h.reshape(1, -1), W_o, b_o.reshape(1, -1))


# speedup vs baseline: 2.6754x; 2.6754x over previous
import functools

import jax
import jax.numpy as jnp
from jax import lax
from jax.experimental import pallas as pl
from jax.experimental.pallas import tpu as pltpu
from jax.experimental.pallas import tpu_sc as plsc


def _sc_gather_words(table_flat, idx_flat):
    n = idx_flat.shape[0]
    info = plsc.get_sparse_core_info()
    nc, ns = info.num_cores, info.num_subcores
    nw = nc * ns
    assert n % nw == 0 and (n // nw) % 8 == 0
    n_per_w = n // nw
    mesh = plsc.VectorSubcoreMesh(core_axis_name="c", subcore_axis_name="s")

    @functools.partial(
        pl.kernel,
        mesh=mesh,
        out_type=jax.ShapeDtypeStruct((n,), jnp.float32),
        compiler_params=pltpu.CompilerParams(use_tc_tiling_on_sc=True),
        scratch_types=[
            pltpu.VMEM((n_per_w,), jnp.int32),
            pltpu.VMEM((n_per_w,), jnp.float32),
            pltpu.SemaphoreType.DMA,
        ],
    )
    def gather_kernel(table_hbm, idx_hbm, out_hbm, idx_v, rows_v, sem):
        wid = lax.axis_index("s") * nc + lax.axis_index("c")
        base = wid * n_per_w
        pltpu.sync_copy(idx_hbm.at[pl.ds(base, n_per_w)], idx_v)
        pltpu.async_copy(table_hbm.at[idx_v], rows_v, sem).wait()
        pltpu.sync_copy(rows_v, out_hbm.at[pl.ds(base, n_per_w)])

    return gather_kernel(table_flat, idx_flat)


def _mlp_t(e, w_h_t, b_h2, w_o, b_o2, vt=4096):
    bsz, wd = e.shape
    h = w_h_t.shape[1]
    v = w_o.shape[0]
    grid = (v + vt - 1) // vt

    def body(e_ref, wh_ref, bh_ref, wo_ref, bo_ref, out_ref, h_ref):
        @pl.when(pl.program_id(0) == 0)
        def _():
            acc = lax.dot_general(
                e_ref[...], wh_ref[...], (((1,), (0,)), ((), ())),
                preferred_element_type=jnp.float32)
            h_ref[...] = jnp.tanh(acc + bh_ref[...])

        ones_row = jnp.ones((1, bsz), jnp.float32)
        out_ref[...] = lax.dot_general(
            wo_ref[...], h_ref[...], (((1,), (1,)), ((), ())),
            preferred_element_type=jnp.float32) + lax.dot_general(
            bo_ref[...], ones_row, (((0,), (0,)), ((), ())),
            preferred_element_type=jnp.float32)

    return pl.pallas_call(
        body,
        grid=(grid,),
        in_specs=[
            pl.BlockSpec((bsz, wd), lambda j: (0, 0)),
            pl.BlockSpec((wd, h), lambda j: (0, 0)),
            pl.BlockSpec((1, h), lambda j: (0, 0)),
            pl.BlockSpec((vt, h), lambda j: (j, 0)),
            pl.BlockSpec((1, vt), lambda j: (0, j)),
        ],
        out_specs=pl.BlockSpec((vt, bsz), lambda j: (j, 0)),
        out_shape=jax.ShapeDtypeStruct((v, bsz), jnp.float32),
        scratch_shapes=[pltpu.VMEM((bsz, h), jnp.float32)],
    )(e, w_h_t, b_h2, w_o, b_o2)


def kernel(x, emb, W_h, b_h, W_o, b_o):
    batch, window = x.shape
    vocab, d = emb.shape
    table_flat = jnp.transpose(emb).reshape(-1)
    idx_flat = (x.reshape(-1, 1).astype(jnp.int32)
                + jnp.arange(d, dtype=jnp.int32)[None, :] * vocab).reshape(-1)
    rows = _sc_gather_words(table_flat, idx_flat)
    e = rows.reshape(batch, window * d)
    out_t = _mlp_t(e, jnp.transpose(W_h), b_h.reshape(1, -1), W_o,
                   b_o.reshape(1, -1))
    return jnp.transpose(out_t)


# speedup vs baseline: 2.7272x; 1.0194x over previous
import functools

import jax
import jax.numpy as jnp
from jax import lax
from jax.experimental import pallas as pl
from jax.experimental.pallas import tpu as pltpu
from jax.experimental.pallas import tpu_sc as plsc

_LANES = 128


def _sc_gather_words(table_flat, idx_flat):
    n = idx_flat.shape[0]
    info = plsc.get_sparse_core_info()
    nc, ns = info.num_cores, info.num_subcores
    nw = nc * ns
    assert n % nw == 0 and (n // nw) % 8 == 0
    n_per_w = n // nw
    mesh = plsc.VectorSubcoreMesh(core_axis_name="c", subcore_axis_name="s")

    @functools.partial(
        pl.kernel,
        mesh=mesh,
        out_type=jax.ShapeDtypeStruct((n,), jnp.float32),
        compiler_params=pltpu.CompilerParams(use_tc_tiling_on_sc=True),
        scratch_types=[
            pltpu.VMEM((n_per_w,), jnp.int32),
            pltpu.VMEM((n_per_w,), jnp.float32),
            pltpu.SemaphoreType.DMA,
        ],
    )
    def gather_kernel(table_hbm, idx_hbm, out_hbm, idx_v, rows_v, sem):
        wid = lax.axis_index("s") * nc + lax.axis_index("c")
        base = wid * n_per_w
        pltpu.sync_copy(idx_hbm.at[pl.ds(base, n_per_w)], idx_v)
        pltpu.async_copy(table_hbm.at[idx_v], rows_v, sem).wait()
        pltpu.sync_copy(rows_v, out_hbm.at[pl.ds(base, n_per_w)])

    return gather_kernel(table_flat, idx_flat)


def _mlp_t(et_c, w_h_t, b_h_col, w_o, b_o2, vt=4096):
    wd, hid = w_h_t.shape
    n_chunks = et_c.shape[0] // wd
    bsz = n_chunks * _LANES
    v = w_o.shape[0]
    grid = (v + vt - 1) // vt

    def body(e_ref, wh_ref, bh_ref, wo_ref, bo_ref, out_ref, h_ref):
        @pl.when(pl.program_id(0) == 0)
        def _():
            for c in range(n_chunks):
                acc = lax.dot_general(
                    wh_ref[...], e_ref[c * wd:(c + 1) * wd, :],
                    (((0,), (0,)), ((), ())),
                    preferred_element_type=jnp.float32)
                h_ref[:, c * _LANES:(c + 1) * _LANES] = jnp.tanh(
                    acc + bh_ref[...])

        ones_row = jnp.ones((1, bsz), jnp.float32)
        out_ref[...] = lax.dot_general(
            wo_ref[...], h_ref[...], (((1,), (0,)), ((), ())),
            preferred_element_type=jnp.float32) + lax.dot_general(
            bo_ref[...], ones_row, (((0,), (0,)), ((), ())),
            preferred_element_type=jnp.float32)

    return pl.pallas_call(
        body,
        grid=(grid,),
        in_specs=[
            pl.BlockSpec((n_chunks * wd, _LANES), lambda j: (0, 0)),
            pl.BlockSpec((wd, hid), lambda j: (0, 0)),
            pl.BlockSpec((hid, 1), lambda j: (0, 0)),
            pl.BlockSpec((vt, hid), lambda j: (j, 0)),
            pl.BlockSpec((1, vt), lambda j: (0, j)),
        ],
        out_specs=pl.BlockSpec((vt, bsz), lambda j: (j, 0)),
        out_shape=jax.ShapeDtypeStruct((v, bsz), jnp.float32),
        scratch_shapes=[pltpu.VMEM((hid, bsz), jnp.float32)],
    )(et_c, w_h_t, b_h_col, w_o, b_o2)


def kernel(x, emb, W_h, b_h, W_o, b_o):
    batch, window = x.shape
    vocab, d = emb.shape
    n_chunks = batch // _LANES
    table_flat = jnp.transpose(emb).reshape(-1)
    xt_c = jnp.transpose(x.astype(jnp.int32)).reshape(
        window, n_chunks, _LANES).transpose(1, 0, 2)
    idx_flat = (xt_c[:, :, None, :]
                + (jnp.arange(d, dtype=jnp.int32) * vocab)[None, None, :, None]
                ).reshape(-1)
    rows = _sc_gather_words(table_flat, idx_flat)
    et_c = rows.reshape(n_chunks * window * d, _LANES)
    out_t = _mlp_t(et_c, jnp.transpose(W_h), b_h.reshape(-1, 1), W_o,
                   b_o.reshape(1, -1))
    return jnp.transpose(out_t)


# speedup vs baseline: 2.7323x; 1.0019x over previous
import functools

import jax
import jax.numpy as jnp
from jax import lax
from jax.experimental import pallas as pl
from jax.experimental.pallas import tpu as pltpu
from jax.experimental.pallas import tpu_sc as plsc

_LANES = 128


def _sc_gather_words(table_flat, idx_flat):
    n = idx_flat.shape[0]
    info = plsc.get_sparse_core_info()
    nc, ns = info.num_cores, info.num_subcores
    nw = nc * ns
    assert n % nw == 0 and (n // nw) % 8 == 0
    n_per_w = n // nw
    mesh = plsc.VectorSubcoreMesh(core_axis_name="c", subcore_axis_name="s")

    @functools.partial(
        pl.kernel,
        mesh=mesh,
        out_type=jax.ShapeDtypeStruct((n,), jnp.float32),
        compiler_params=pltpu.CompilerParams(use_tc_tiling_on_sc=True),
        scratch_types=[
            pltpu.VMEM((n_per_w,), jnp.int32),
            pltpu.VMEM((n_per_w,), jnp.float32),
            pltpu.SemaphoreType.DMA,
        ],
    )
    def gather_kernel(table_hbm, idx_hbm, out_hbm, idx_v, rows_v, sem):
        wid = lax.axis_index("s") * nc + lax.axis_index("c")
        base = wid * n_per_w
        pltpu.sync_copy(idx_hbm.at[pl.ds(base, n_per_w)], idx_v)
        pltpu.async_copy(table_hbm.at[idx_v], rows_v, sem).wait()
        pltpu.sync_copy(rows_v, out_hbm.at[pl.ds(base, n_per_w)])

    return gather_kernel(table_flat, idx_flat)


def _mlp_t(et_flat, w_h_t, b_h_col, w_o, b_o2, vt=4096):
    wd, hid = w_h_t.shape
    n_chunks = et_flat.shape[0] // (wd * _LANES)
    bsz = n_chunks * _LANES
    v = w_o.shape[0]
    grid = (v + vt - 1) // vt

    def body(e_ref, wh_ref, bh_ref, wo_ref, bo_ref, out_ref, h_ref):
        @pl.when(pl.program_id(0) == 0)
        def _():
            for c in range(n_chunks):
                chunk = e_ref[pl.ds(c * wd * _LANES, wd * _LANES)].reshape(
                    wd, _LANES)
                acc = lax.dot_general(
                    wh_ref[...], chunk, (((0,), (0,)), ((), ())),
                    preferred_element_type=jnp.float32)
                h_ref[:, c * _LANES:(c + 1) * _LANES] = jnp.tanh(
                    acc + bh_ref[...])

        ones_row = jnp.ones((1, bsz), jnp.float32)
        out_ref[...] = lax.dot_general(
            wo_ref[...], h_ref[...], (((1,), (0,)), ((), ())),
            preferred_element_type=jnp.float32) + lax.dot_general(
            bo_ref[...], ones_row, (((0,), (0,)), ((), ())),
            preferred_element_type=jnp.float32)

    return pl.pallas_call(
        body,
        grid=(grid,),
        in_specs=[
            pl.BlockSpec((n_chunks * wd * _LANES,), lambda j: (0,)),
            pl.BlockSpec((wd, hid), lambda j: (0, 0)),
            pl.BlockSpec((hid, 1), lambda j: (0, 0)),
            pl.BlockSpec((vt, hid), lambda j: (j, 0)),
            pl.BlockSpec((1, vt), lambda j: (0, j)),
        ],
        out_specs=pl.BlockSpec((vt, bsz), lambda j: (j, 0)),
        out_shape=jax.ShapeDtypeStruct((v, bsz), jnp.float32),
        scratch_shapes=[pltpu.VMEM((hid, bsz), jnp.float32)],
    )(et_flat, w_h_t, b_h_col, w_o, b_o2)


def kernel(x, emb, W_h, b_h, W_o, b_o):
    batch, window = x.shape
    vocab, d = emb.shape
    n_chunks = batch // _LANES
    table_flat = jnp.transpose(emb).reshape(-1)
    xt_c = jnp.transpose(x.astype(jnp.int32)).reshape(
        window, n_chunks, _LANES).transpose(1, 0, 2)
    idx_flat = (xt_c[:, :, None, :]
                + (jnp.arange(d, dtype=jnp.int32) * vocab)[None, None, :, None]
                ).reshape(-1)
    rows = _sc_gather_words(table_flat, idx_flat)
    out_t = _mlp_t(rows, jnp.transpose(W_h), b_h.reshape(-1, 1), W_o,
                   b_o.reshape(1, -1))
    return jnp.transpose(out_t)
